# baseline (speedup 1.0000x reference)
import functools

import jax
import jax.numpy as jnp
from jax import lax
from jax.experimental import pallas as pl
from jax.experimental.pallas import tpu as pltpu
from jax.experimental.pallas import tpu_sc as plsc

N = 10000
E = 320000
D = 128

NSPLIT = 5
EH = E // NSPLIT
NC = 2
NS = 16
NW = NC * NS
PER_W = EH // NW
CH = 40
NBUF = 5
NCH = PER_W // CH
NROUND = NCH // NBUF
CHS = 40
NCHS = PER_W // CHS
NBUFS = 5
NROUNDS = NCHS // NBUFS
N_PAD = 10240
STRIPE = N_PAD // NS

_sc_mesh = lambda: plsc.VectorSubcoreMesh(core_axis_name="c", subcore_axis_name="s")


def _gather_body(part, h_hbm, src_hbm, dst_hbm, hs_hbm, hd_hbm,
                 idx_s, idx_d, bufs, sems_g, sems_w):
    c = lax.axis_index("c")
    s = lax.axis_index("s")
    base = part * EH + (s * NC + c) * PER_W

    pltpu.sync_copy(src_hbm.at[pl.ds(base, PER_W)], idx_s)
    pltpu.sync_copy(dst_hbm.at[pl.ds(base, PER_W)], idx_d)

    idx = (idx_s, idx_d)
    outs = (hs_hbm, hd_hbm)
    obase = (s * NC + c) * PER_W

    def fire(dirn, b, ch):
        pltpu.async_copy(h_hbm.at[idx[dirn].at[pl.ds(ch * CH, CH)]],
                         bufs[dirn][b], sems_g[dirn][b])

    def wait_g(dirn, b):
        pltpu.make_async_copy(h_hbm.at[idx[dirn].at[pl.ds(0, CH)]],
                              bufs[dirn][b], sems_g[dirn][b]).wait()

    def fire_w(dirn, b, ch):
        pltpu.async_copy(bufs[dirn][b],
                         outs[dirn].at[pl.ds(obase + ch * CH, CH)],
                         sems_w[dirn][b])

    def wait_w(dirn, b):
        pltpu.make_async_copy(bufs[dirn][b],
                              outs[dirn].at[pl.ds(obase, CH)],
                              sems_w[dirn][b]).wait()

    for dirn in range(2):
        for b in range(NBUF):
            fire(dirn, b, b)

    def round_body(m, carry):
        for dirn in range(2):
            for b in range(NBUF):
                ch = m * NBUF + b
                wait_g(dirn, b)
                fire_w(dirn, b, ch)

        @pl.when(m < NROUND - 1)
        def _():
            for dirn in range(2):
                for b in range(NBUF):
                    wait_w(dirn, b)
                    fire(dirn, b, (m + 1) * NBUF + b)
        return carry

    lax.fori_loop(0, NROUND, round_body, 0)
    for dirn in range(2):
        for b in range(NBUF):
            wait_w(dirn, b)


def _sc_gather(h, src, dst, part):
    f = functools.partial(
        pl.kernel,
        out_type=(jax.ShapeDtypeStruct((EH, D), jnp.float32),
                  jax.ShapeDtypeStruct((EH, D), jnp.float32)),
        mesh=_sc_mesh(),
        scratch_types=[
            pltpu.VMEM((PER_W,), jnp.int32),
            pltpu.VMEM((PER_W,), jnp.int32),
            tuple(tuple(pltpu.VMEM((CH, D), jnp.float32) for _ in range(NBUF))
                  for _ in range(2)),
            tuple(tuple(pltpu.SemaphoreType.DMA for _ in range(NBUF))
                  for _ in range(2)),
            tuple(tuple(pltpu.SemaphoreType.DMA for _ in range(NBUF))
                  for _ in range(2)),
        ],
    )(functools.partial(_gather_body, part))
    return f(h, src, dst)


BE = 3200
HB = EH // BE


def _gelu(x):
    return 0.5 * x * (1.0 + lax.erf(x * 0.7071067811865476))


def _edge_body(hs_ref, hd_ref, e_ref, w1a_ref, w1b_ref, w1c_ref, b1_ref,
               w2_ref, b2_ref, eg_ref, ebb_ref, gwm_ref, gb_ref, ones_ref,
               enew_ref, msg_ref):
    e = e_ref[...]
    u = (jnp.dot(hs_ref[...].astype(jnp.bfloat16), w1a_ref[...],
                 preferred_element_type=jnp.float32)
         + jnp.dot(hd_ref[...].astype(jnp.bfloat16), w1b_ref[...],
                   preferred_element_type=jnp.float32)
         + jnp.dot(e.astype(jnp.bfloat16), w1c_ref[...],
                   preferred_element_type=jnp.float32)
         + b1_ref[...])
    g = _gelu(u)
    r = e + jnp.dot(g.astype(jnp.bfloat16), w2_ref[...],
                    preferred_element_type=jnp.float32) + b2_ref[...]
    rb = r.astype(jnp.bfloat16)
    m = jnp.dot(rb, ones_ref[...], preferred_element_type=jnp.float32) * (1.0 / D)
    sq = (r * r).astype(jnp.bfloat16)
    ex2 = jnp.dot(sq, ones_ref[...], preferred_element_type=jnp.float32) * (1.0 / D)
    v = ex2 - m * m
    ctr = r - m
    en = ctr * lax.rsqrt(v + 1e-5) * eg_ref[...] + ebb_ref[...]
    logit = jnp.dot(en.astype(jnp.bfloat16), gwm_ref[...],
                    preferred_element_type=jnp.float32) + gb_ref[...]
    gate = jax.nn.sigmoid(logit)
    enew_ref[...] = en
    msg_ref[...] = gate * en


def _edge_body_alias(hs_ref, hd_ref, e_ref, w1a_ref, w1b_ref, w1c_ref,
                     b1_ref, w2_ref, b2_ref, eg_ref, ebb_ref, gwm_ref,
                     gb_ref, ones_ref, ebuf_ref, enew_ref, msg_ref):
    del ebuf_ref
    _edge_body(hs_ref, hd_ref, e_ref, w1a_ref, w1b_ref, w1c_ref, b1_ref,
               w2_ref, b2_ref, eg_ref, ebb_ref, gwm_ref, gb_ref, ones_ref,
               enew_ref, msg_ref)


def _tc_edge(hs, hd, e, eW1, eb1, eW2, eb2, e_g, e_b, gW, gb, part, ebuf):
    bf = jnp.bfloat16
    w1a = eW1[:D].astype(bf)
    w1b = eW1[D:2 * D].astype(bf)
    w1c = eW1[2 * D:].astype(bf)
    w2 = eW2.astype(bf)
    gwm = jnp.broadcast_to(gW, (D, D)).astype(bf)
    ones = jnp.ones((D, D), bf)
    full = lambda shape: pl.BlockSpec(shape, lambda i: (0, 0))
    blk = pl.BlockSpec((BE, D), lambda i: (i, 0))
    off = pl.BlockSpec((BE, D), lambda i: (i + part * HB, 0))
    enew, msg = pl.pallas_call(
        _edge_body if ebuf is None else _edge_body_alias,
        grid=(HB,),
        in_specs=[blk, blk, off,
                  full((D, 2 * D)), full((D, 2 * D)), full((D, 2 * D)),
                  full((1, 2 * D)), full((2 * D, D)), full((1, D)),
                  full((1, D)), full((1, D)), full((D, D)), full((1, 1)),
                  full((D, D))] + ([off] if ebuf is not None else []),
        out_specs=[off, blk],
        out_shape=[jax.ShapeDtypeStruct((E, D), jnp.float32),
                   jax.ShapeDtypeStruct((EH, D), jnp.float32)],
        input_output_aliases={14: 0} if ebuf is not None else {},
        compiler_params=pltpu.CompilerParams(
            dimension_semantics=("arbitrary",)),
    )(*([hs, hd, e, w1a, w1b, w1c, eb1.reshape(1, -1), w2, eb2.reshape(1, -1),
         e_g.reshape(1, -1), e_b.reshape(1, -1), gwm, gb.reshape(1, 1), ones]
        + ([ebuf] if ebuf is not None else [])))
    return enew, msg


def _scatter_body(part, msg_hbm, dst_hbm, zeros_hbm, out_hbm,
                  idxs, bufs, sems_i, sems_l, sems_a, agg_sh):
    c = lax.axis_index("c")
    s = lax.axis_index("s")
    base = part * EH + (s * NC + c) * PER_W
    mbase = (s * NC + c) * PER_W
    pltpu.sync_copy(zeros_hbm, agg_sh.at[pl.ds(s * STRIPE, STRIPE)])
    plsc.subcore_barrier()

    def fire_l(b, ch):
        pltpu.async_copy(dst_hbm.at[pl.ds(base + ch * CHS, CHS)],
                         idxs[b], sems_i[b])
        pltpu.async_copy(msg_hbm.at[pl.ds(mbase + ch * CHS, CHS)],
                         bufs[b], sems_l[b])

    def wait_l(b):
        pltpu.make_async_copy(dst_hbm.at[pl.ds(base, CHS)],
                              idxs[b], sems_i[b]).wait()
        pltpu.make_async_copy(msg_hbm.at[pl.ds(mbase, CHS)],
                              bufs[b], sems_l[b]).wait()

    def fire_a(b):
        pltpu.async_copy(bufs[b], agg_sh.at[idxs[b]], sems_a[b], add=True)

    def wait_a(b):
        pltpu.make_async_copy(bufs[b], agg_sh.at[idxs[b]],
                              sems_a[b]).wait()

    for b in range(NBUFS):
        fire_l(b, b)

    def round_body(m, carry):
        for b in range(NBUFS):
            wait_l(b)
            fire_a(b)

        @pl.when(m < NROUNDS - 1)
        def _():
            for b in range(NBUFS):
                wait_a(b)
                fire_l(b, (m + 1) * NBUFS + b)
        return carry

    lax.fori_loop(0, NROUNDS, round_body, 0)
    for b in range(NBUFS):
        wait_a(b)
    plsc.subcore_barrier()
    pltpu.sync_copy(agg_sh.at[pl.ds(s * STRIPE, STRIPE)],
                    out_hbm.at[pl.ds(c * N_PAD + s * STRIPE, STRIPE)])


def _sc_scatter(msg, dst, part):
    zeros = jnp.zeros((STRIPE, D), jnp.float32)
    f = functools.partial(
        pl.kernel,
        out_type=jax.ShapeDtypeStruct((2 * N_PAD, D), jnp.float32),
        mesh=_sc_mesh(),
        scratch_types=[
            tuple(pltpu.VMEM((CHS,), jnp.int32) for _ in range(NBUFS)),
            tuple(pltpu.VMEM((CHS, D), jnp.float32) for _ in range(NBUFS)),
            tuple(pltpu.SemaphoreType.DMA for _ in range(NBUFS)),
            tuple(pltpu.SemaphoreType.DMA for _ in range(NBUFS)),
            tuple(pltpu.SemaphoreType.DMA for _ in range(NBUFS)),
            pltpu.VMEM_SHARED((N_PAD, D), jnp.float32),
        ],
    )(functools.partial(_scatter_body, part))
    return f(msg, dst, zeros)


BN = 1000


def _nodeA_body(h_ref, *refs):
    (p_refs, (w1a_ref, w1b_ref, b1_ref, w2_ref, b2_ref, ng_ref, nbb_ref,
              hnew_ref, csum_ref)) = refs[:2 * NSPLIT], refs[2 * NSPLIT:]
    i = pl.program_id(0)
    h = h_ref[...]
    agg = p_refs[0][...]
    for p_ref in p_refs[1:]:
        agg = agg + p_ref[...]
    u = (jnp.dot(h, w1a_ref[...], preferred_element_type=jnp.float32)
         + jnp.dot(agg, w1b_ref[...], preferred_element_type=jnp.float32)
         + b1_ref[...])
    g = _gelu(u)
    r = h + jnp.dot(g, w2_ref[...], preferred_element_type=jnp.float32) + b2_ref[...]
    m = jnp.mean(r, axis=-1, keepdims=True)
    ctr = r - m
    v = jnp.mean(ctr * ctr, axis=-1, keepdims=True)
    hn = ctr * lax.rsqrt(v + 1e-5) * ng_ref[...] + nbb_ref[...]
    hnew_ref[...] = hn

    @pl.when(i == 0)
    def _():
        csum_ref[...] = jnp.zeros_like(csum_ref)

    csum_ref[...] += jnp.sum(hn, axis=0, keepdims=True)


def _nodeB_body(hn_ref, csum_ref, glw_ref, glb_ref, out_ref):
    ctx = csum_ref[0:1, :] * (1.0 / N)
    delta = jnp.dot(ctx, glw_ref[...], preferred_element_type=jnp.float32) + glb_ref[...]
    out_ref[...] = hn_ref[...] + delta


def _tc_node(h, parts, nW1, nb1, nW2, nb2, n_g, n_b, glW, glb):
    w1a, w1b = nW1[:D], nW1[D:]
    ps = []
    for part in parts:
        ps += [part[:N], part[N_PAD:N_PAD + N]]
    full = lambda shape: pl.BlockSpec(shape, lambda i: (0, 0))
    blk = pl.BlockSpec((BN, D), lambda i: (i, 0))
    hn, csum = pl.pallas_call(
        _nodeA_body,
        grid=(N // BN,),
        in_specs=[blk] + [blk] * (2 * NSPLIT) +
                 [full((D, 2 * D)), full((D, 2 * D)), full((1, 2 * D)),
                  full((2 * D, D)), full((1, D)), full((1, D)), full((1, D))],
        out_specs=[blk, full((8, D))],
        out_shape=[jax.ShapeDtypeStruct((N, D), jnp.float32),
                   jax.ShapeDtypeStruct((8, D), jnp.float32)],
        compiler_params=pltpu.CompilerParams(
            dimension_semantics=("arbitrary",)),
    )(h, *ps, w1a, w1b, nb1.reshape(1, -1), nW2,
      nb2.reshape(1, -1), n_g.reshape(1, -1), n_b.reshape(1, -1))
    h_out = pl.pallas_call(
        _nodeB_body,
        grid=(N // BN,),
        in_specs=[blk, full((8, D)), full((D, D)), full((1, D))],
        out_specs=blk,
        out_shape=jax.ShapeDtypeStruct((N, D), jnp.float32),
        compiler_params=pltpu.CompilerParams(
            dimension_semantics=("arbitrary",)),
    )(hn, csum, glW, glb.reshape(1, -1))
    return h_out


def kernel(h, e, eW1, eb1, eW2, eb2, e_g, e_b, gW, gb, nW1, nb1, nW2, nb2,
           n_g, n_b, glW, glb, edge_index):
    src = edge_index[0]
    dst = edge_index[1]
    gathers = [_sc_gather(h, src, dst, p) for p in range(NSPLIT)]
    parts = []
    ebuf = None
    for p in range(NSPLIT):
        hs, hd = gathers[p]
        ebuf, msg = _tc_edge(hs, hd, e, eW1, eb1, eW2, eb2, e_g, e_b, gW, gb,
                             p, ebuf)
        parts.append(_sc_scatter(msg, dst, p))
    h_out = _tc_node(h, parts, nW1, nb1, nW2, nb2, n_g, n_b, glW, glb)
    return (h_out, ebuf)


# speedup vs baseline: 1.1128x; 1.1128x over previous
import functools

import jax
import jax.numpy as jnp
from jax import lax
from jax.experimental import pallas as pl
from jax.experimental.pallas import tpu as pltpu
from jax.experimental.pallas import tpu_sc as plsc

N = 10000
E = 320000
D = 128

NSPLIT = 2
EH = E // NSPLIT
NC = 2
NS = 16
NW = NC * NS
PER_W = EH // NW
CH = 40
NBUF = 5
NCH = PER_W // CH
NROUND = NCH // NBUF
CHS = 40
NCHS = PER_W // CHS
NBUFS = 5
NROUNDS = NCHS // NBUFS
N_PAD = 10240
STRIPE = N_PAD // NS

_sc_mesh = lambda: plsc.VectorSubcoreMesh(core_axis_name="c", subcore_axis_name="s")


BP = 1000


def _pack_halves(x):
    hi = x[:, :D].astype(jnp.bfloat16).astype(jnp.float32)
    lo = x[:, D:].astype(jnp.bfloat16).astype(jnp.float32)
    hi_i = lax.bitcast_convert_type(hi, jnp.int32)
    lo_i = lax.shift_right_logical(lax.bitcast_convert_type(lo, jnp.int32), 16)
    return lax.bitwise_or(hi_i, lo_i)


def _pre_body(h_ref, w1a_ref, w1b_ref, a_ref, b_ref):
    hb = h_ref[...].astype(jnp.bfloat16)
    a = jnp.dot(hb, w1a_ref[...], preferred_element_type=jnp.float32)
    b = jnp.dot(hb, w1b_ref[...], preferred_element_type=jnp.float32)
    a_ref[...] = _pack_halves(a)
    b_ref[...] = _pack_halves(b)


def _tc_pre(h, eW1):
    bf = jnp.bfloat16
    w1a = eW1[:D].astype(bf)
    w1b = eW1[D:2 * D].astype(bf)
    full = lambda shape: pl.BlockSpec(shape, lambda i: (0, 0))
    blk = pl.BlockSpec((BP, D), lambda i: (i, 0))
    return pl.pallas_call(
        _pre_body,
        grid=(N // BP,),
        in_specs=[blk, full((D, 2 * D)), full((D, 2 * D))],
        out_specs=[blk, blk],
        out_shape=[jax.ShapeDtypeStruct((N, D), jnp.int32),
                   jax.ShapeDtypeStruct((N, D), jnp.int32)],
        compiler_params=pltpu.CompilerParams(
            dimension_semantics=("arbitrary",)),
    )(h, w1a, w1b)


def _gather_body(part, a_hbm, b_hbm, src_hbm, dst_hbm, as_hbm, bs_hbm,
                 idx_s, idx_d, bufs, sems_g, sems_w):
    c = lax.axis_index("c")
    s = lax.axis_index("s")
    base = part * EH + (s * NC + c) * PER_W

    pltpu.sync_copy(src_hbm.at[pl.ds(base, PER_W)], idx_s)
    pltpu.sync_copy(dst_hbm.at[pl.ds(base, PER_W)], idx_d)

    idx = (idx_s, idx_d)
    tabs = (a_hbm, b_hbm)
    outs = (as_hbm, bs_hbm)
    obase = (s * NC + c) * PER_W

    def fire(dirn, b, ch):
        pltpu.async_copy(tabs[dirn].at[idx[dirn].at[pl.ds(ch * CH, CH)]],
                         bufs[dirn][b], sems_g[dirn][b])

    def wait_g(dirn, b):
        pltpu.make_async_copy(tabs[dirn].at[idx[dirn].at[pl.ds(0, CH)]],
                              bufs[dirn][b], sems_g[dirn][b]).wait()

    def fire_w(dirn, b, ch):
        pltpu.async_copy(bufs[dirn][b],
                         outs[dirn].at[pl.ds(obase + ch * CH, CH)],
                         sems_w[dirn][b])

    def wait_w(dirn, b):
        pltpu.make_async_copy(bufs[dirn][b],
                              outs[dirn].at[pl.ds(obase, CH)],
                              sems_w[dirn][b]).wait()

    for dirn in range(2):
        for b in range(NBUF):
            fire(dirn, b, b)

    def round_body(m, carry):
        for dirn in range(2):
            for b in range(NBUF):
                ch = m * NBUF + b
                wait_g(dirn, b)
                fire_w(dirn, b, ch)

        @pl.when(m < NROUND - 1)
        def _():
            for dirn in range(2):
                for b in range(NBUF):
                    wait_w(dirn, b)
                    fire(dirn, b, (m + 1) * NBUF + b)
        return carry

    lax.fori_loop(0, NROUND, round_body, 0)
    for dirn in range(2):
        for b in range(NBUF):
            wait_w(dirn, b)


def _sc_gather(a, b, src, dst, part):
    f = functools.partial(
        pl.kernel,
        out_type=(jax.ShapeDtypeStruct((EH, D), jnp.int32),
                  jax.ShapeDtypeStruct((EH, D), jnp.int32)),
        mesh=_sc_mesh(),
        scratch_types=[
            pltpu.VMEM((PER_W,), jnp.int32),
            pltpu.VMEM((PER_W,), jnp.int32),
            tuple(tuple(pltpu.VMEM((CH, D), jnp.int32) for _ in range(NBUF))
                  for _ in range(2)),
            tuple(tuple(pltpu.SemaphoreType.DMA for _ in range(NBUF))
                  for _ in range(2)),
            tuple(tuple(pltpu.SemaphoreType.DMA for _ in range(NBUF))
                  for _ in range(2)),
        ],
    )(functools.partial(_gather_body, part))
    return f(a, b, src, dst)


BE = 3200
HB = EH // BE


def _gelu(x):
    return 0.5 * x * (1.0 + lax.erf(x * 0.7071067811865476))


def _unpack_hi(x):
    return lax.bitcast_convert_type(
        lax.bitwise_and(x, jnp.int32(-65536)), jnp.float32)


def _unpack_lo(x):
    return lax.bitcast_convert_type(lax.shift_left(x, 16), jnp.float32)


def _edge_body(as_ref, bs_ref, e_ref, w1c_ref, b1_ref,
               w2a_ref, w2b_ref, b2_ref, eg_ref, ebb_ref, gwm_ref, gb_ref,
               ones_ref, enew_ref, msg_ref):
    e = e_ref[...]
    a32 = as_ref[...]
    b32 = bs_ref[...]
    ec = jnp.dot(e.astype(jnp.bfloat16), w1c_ref[...],
                 preferred_element_type=jnp.float32) + b1_ref[...]
    u1 = _unpack_hi(a32) + _unpack_hi(b32) + ec[:, :D]
    u2 = _unpack_lo(a32) + _unpack_lo(b32) + ec[:, D:]
    g1 = _gelu(u1)
    g2 = _gelu(u2)
    r = (e + jnp.dot(g1.astype(jnp.bfloat16), w2a_ref[...],
                     preferred_element_type=jnp.float32)
         + jnp.dot(g2.astype(jnp.bfloat16), w2b_ref[...],
                   preferred_element_type=jnp.float32) + b2_ref[...])
    rb = r.astype(jnp.bfloat16)
    m = jnp.dot(rb, ones_ref[...], preferred_element_type=jnp.float32) * (1.0 / D)
    sq = (r * r).astype(jnp.bfloat16)
    ex2 = jnp.dot(sq, ones_ref[...], preferred_element_type=jnp.float32) * (1.0 / D)
    v = ex2 - m * m
    ctr = r - m
    en = ctr * lax.rsqrt(v + 1e-5) * eg_ref[...] + ebb_ref[...]
    logit = jnp.dot(en.astype(jnp.bfloat16), gwm_ref[...],
                    preferred_element_type=jnp.float32) + gb_ref[...]
    gate = jax.nn.sigmoid(logit)
    enew_ref[...] = en
    msg_ref[...] = gate * en


def _edge_body_alias(as_ref, bs_ref, e_ref, w1c_ref, b1_ref, w2a_ref,
                     w2b_ref, b2_ref, eg_ref, ebb_ref, gwm_ref, gb_ref,
                     ones_ref, ebuf_ref, enew_ref, msg_ref):
    del ebuf_ref
    _edge_body(as_ref, bs_ref, e_ref, w1c_ref, b1_ref, w2a_ref, w2b_ref,
               b2_ref, eg_ref, ebb_ref, gwm_ref, gb_ref, ones_ref,
               enew_ref, msg_ref)


def _tc_edge(asg, bsg, e, eW1, eb1, eW2, eb2, e_g, e_b, gW, gb, part, ebuf):
    bf = jnp.bfloat16
    w1c = eW1[2 * D:].astype(bf)
    w2a = eW2[:D].astype(bf)
    w2b = eW2[D:].astype(bf)
    gwm = jnp.broadcast_to(gW, (D, D)).astype(bf)
    ones = jnp.ones((D, D), bf)
    full = lambda shape: pl.BlockSpec(shape, lambda i: (0, 0))
    blk = pl.BlockSpec((BE, D), lambda i: (i, 0))
    off = pl.BlockSpec((BE, D), lambda i: (i + part * HB, 0))
    enew, msg = pl.pallas_call(
        _edge_body if ebuf is None else _edge_body_alias,
        grid=(HB,),
        in_specs=[blk, blk, off,
                  full((D, 2 * D)), full((1, 2 * D)),
                  full((D, D)), full((D, D)), full((1, D)),
                  full((1, D)), full((1, D)), full((D, D)), full((1, 1)),
                  full((D, D))] + ([off] if ebuf is not None else []),
        out_specs=[off, blk],
        out_shape=[jax.ShapeDtypeStruct((E, D), jnp.float32),
                   jax.ShapeDtypeStruct((EH, D), jnp.float32)],
        input_output_aliases={13: 0} if ebuf is not None else {},
        compiler_params=pltpu.CompilerParams(
            dimension_semantics=("arbitrary",)),
    )(*([asg, bsg, e, w1c, eb1.reshape(1, -1), w2a, w2b, eb2.reshape(1, -1),
         e_g.reshape(1, -1), e_b.reshape(1, -1), gwm, gb.reshape(1, 1), ones]
        + ([ebuf] if ebuf is not None else [])))
    return enew, msg


def _scatter_body(part, msg_hbm, dst_hbm, zeros_hbm, out_hbm,
                  idxs, bufs, sems_i, sems_l, sems_a, agg_sh):
    c = lax.axis_index("c")
    s = lax.axis_index("s")
    base = part * EH + (s * NC + c) * PER_W
    mbase = (s * NC + c) * PER_W
    pltpu.sync_copy(zeros_hbm, agg_sh.at[pl.ds(s * STRIPE, STRIPE)])
    plsc.subcore_barrier()

    def fire_l(b, ch):
        pltpu.async_copy(dst_hbm.at[pl.ds(base + ch * CHS, CHS)],
                         idxs[b], sems_i[b])
        pltpu.async_copy(msg_hbm.at[pl.ds(mbase + ch * CHS, CHS)],
                         bufs[b], sems_l[b])

    def wait_l(b):
        pltpu.make_async_copy(dst_hbm.at[pl.ds(base, CHS)],
                              idxs[b], sems_i[b]).wait()
        pltpu.make_async_copy(msg_hbm.at[pl.ds(mbase, CHS)],
                              bufs[b], sems_l[b]).wait()

    def fire_a(b):
        pltpu.async_copy(bufs[b], agg_sh.at[idxs[b]], sems_a[b], add=True)

    def wait_a(b):
        pltpu.make_async_copy(bufs[b], agg_sh.at[idxs[b]],
                              sems_a[b]).wait()

    for b in range(NBUFS):
        fire_l(b, b)

    def round_body(m, carry):
        for b in range(NBUFS):
            wait_l(b)
            fire_a(b)

        @pl.when(m < NROUNDS - 1)
        def _():
            for b in range(NBUFS):
                wait_a(b)
                fire_l(b, (m + 1) * NBUFS + b)
        return carry

    lax.fori_loop(0, NROUNDS, round_body, 0)
    for b in range(NBUFS):
        wait_a(b)
    plsc.subcore_barrier()
    pltpu.sync_copy(agg_sh.at[pl.ds(s * STRIPE, STRIPE)],
                    out_hbm.at[pl.ds(c * N_PAD + s * STRIPE, STRIPE)])


def _sc_scatter(msg, dst, part):
    zeros = jnp.zeros((STRIPE, D), jnp.float32)
    f = functools.partial(
        pl.kernel,
        out_type=jax.ShapeDtypeStruct((2 * N_PAD, D), jnp.float32),
        mesh=_sc_mesh(),
        scratch_types=[
            tuple(pltpu.VMEM((CHS,), jnp.int32) for _ in range(NBUFS)),
            tuple(pltpu.VMEM((CHS, D), jnp.float32) for _ in range(NBUFS)),
            tuple(pltpu.SemaphoreType.DMA for _ in range(NBUFS)),
            tuple(pltpu.SemaphoreType.DMA for _ in range(NBUFS)),
            tuple(pltpu.SemaphoreType.DMA for _ in range(NBUFS)),
            pltpu.VMEM_SHARED((N_PAD, D), jnp.float32),
        ],
    )(functools.partial(_scatter_body, part))
    return f(msg, dst, zeros)


BN = 1000


def _nodeA_body(h_ref, *refs):
    (p_refs, (w1a_ref, w1b_ref, b1_ref, w2_ref, b2_ref, ng_ref, nbb_ref,
              hnew_ref, csum_ref)) = refs[:2 * NSPLIT], refs[2 * NSPLIT:]
    i = pl.program_id(0)
    h = h_ref[...]
    agg = p_refs[0][...]
    for p_ref in p_refs[1:]:
        agg = agg + p_ref[...]
    u = (jnp.dot(h, w1a_ref[...], preferred_element_type=jnp.float32)
         + jnp.dot(agg, w1b_ref[...], preferred_element_type=jnp.float32)
         + b1_ref[...])
    g = _gelu(u)
    r = h + jnp.dot(g, w2_ref[...], preferred_element_type=jnp.float32) + b2_ref[...]
    m = jnp.mean(r, axis=-1, keepdims=True)
    ctr = r - m
    v = jnp.mean(ctr * ctr, axis=-1, keepdims=True)
    hn = ctr * lax.rsqrt(v + 1e-5) * ng_ref[...] + nbb_ref[...]
    hnew_ref[...] = hn

    @pl.when(i == 0)
    def _():
        csum_ref[...] = jnp.zeros_like(csum_ref)

    csum_ref[...] += jnp.sum(hn, axis=0, keepdims=True)


def _nodeB_body(hn_ref, csum_ref, glw_ref, glb_ref, out_ref):
    ctx = csum_ref[0:1, :] * (1.0 / N)
    delta = jnp.dot(ctx, glw_ref[...], preferred_element_type=jnp.float32) + glb_ref[...]
    out_ref[...] = hn_ref[...] + delta


def _tc_node(h, parts, nW1, nb1, nW2, nb2, n_g, n_b, glW, glb):
    w1a, w1b = nW1[:D], nW1[D:]
    ps = []
    for part in parts:
        ps += [part[:N], part[N_PAD:N_PAD + N]]
    full = lambda shape: pl.BlockSpec(shape, lambda i: (0, 0))
    blk = pl.BlockSpec((BN, D), lambda i: (i, 0))
    hn, csum = pl.pallas_call(
        _nodeA_body,
        grid=(N // BN,),
        in_specs=[blk] + [blk] * (2 * NSPLIT) +
                 [full((D, 2 * D)), full((D, 2 * D)), full((1, 2 * D)),
                  full((2 * D, D)), full((1, D)), full((1, D)), full((1, D))],
        out_specs=[blk, full((8, D))],
        out_shape=[jax.ShapeDtypeStruct((N, D), jnp.float32),
                   jax.ShapeDtypeStruct((8, D), jnp.float32)],
        compiler_params=pltpu.CompilerParams(
            dimension_semantics=("arbitrary",)),
    )(h, *ps, w1a, w1b, nb1.reshape(1, -1), nW2,
      nb2.reshape(1, -1), n_g.reshape(1, -1), n_b.reshape(1, -1))
    h_out = pl.pallas_call(
        _nodeB_body,
        grid=(N // BN,),
        in_specs=[blk, full((8, D)), full((D, D)), full((1, D))],
        out_specs=blk,
        out_shape=jax.ShapeDtypeStruct((N, D), jnp.float32),
        compiler_params=pltpu.CompilerParams(
            dimension_semantics=("arbitrary",)),
    )(hn, csum, glW, glb.reshape(1, -1))
    return h_out


def kernel(h, e, eW1, eb1, eW2, eb2, e_g, e_b, gW, gb, nW1, nb1, nW2, nb2,
           n_g, n_b, glW, glb, edge_index):
    src = edge_index[0]
    dst = edge_index[1]
    a_pk, b_pk = _tc_pre(h, eW1)
    gathers = [_sc_gather(a_pk, b_pk, src, dst, p) for p in range(NSPLIT)]
    parts = []
    ebuf = None
    for p in range(NSPLIT):
        asg, bsg = gathers[p]
        ebuf, msg = _tc_edge(asg, bsg, e, eW1, eb1, eW2, eb2, e_g, e_b,
                             gW, gb, p, ebuf)
        parts.append(_sc_scatter(msg, dst, p))
    h_out = _tc_node(h, parts, nW1, nb1, nW2, nb2, n_g, n_b, glW, glb)
    return (h_out, ebuf)
